# Initial kernel scaffold; baseline (speedup 1.0000x reference)
import jax
import jax.numpy as jnp
from jax.experimental import pallas as pl


def kernel(state, adj, W1, b1, W2, b2, fW1, fb1, fW2, fb2, fW3, fb3):
    raise NotImplementedError("write your pallas kernel here")


# speedup vs baseline: 73.7536x; 73.7536x over previous
import functools

import jax
import jax.numpy as jnp
from jax import lax
from jax.experimental import pallas as pl
from jax.experimental.pallas import tpu as pltpu, tpu_sc as plsc

N = 10000
E = 320000
D = 128
G = 441
F1 = 256
F2 = 256
A = 10

NPAD = 10240
GP = 512
SEG = NPAD // 16
NSUB = 16
NCORE = 2
ECT1 = E // 32
ECT2 = E // 16
CH = 2000
C = 56
CROWS = 64
MCAP = NPAD // 2 + 64
FLUSH = 512
ACAP = FLUSH + 64

_mesh = plsc.VectorSubcoreMesh(core_axis_name="c", subcore_axis_name="s",
                               num_cores=NCORE, num_subcores=NSUB)
_sc_params = pltpu.CompilerParams(needs_layout_passes=False)

_I16Z = functools.partial(jnp.zeros, (16,))


def _zero_ref(ref, n16):
    def body(g, _):
        ref[pl.ds(g * 16, 16)] = jnp.zeros((16,), ref.dtype)
        return 0
    lax.fori_loop(0, n16, body, 0)


def _popcount(m):
    return plsc.all_reduce_population_count(m)[0]


@functools.partial(
    pl.kernel,
    out_type=(jax.ShapeDtypeStruct((NCORE, NPAD), jnp.int32),
              jax.ShapeDtypeStruct((NCORE, NPAD), jnp.int32)),
    mesh=_mesh,
    scratch_types=[
        pltpu.VMEM((CH,), jnp.int32),
        pltpu.VMEM((CH,), jnp.int32),
        pltpu.VMEM((NPAD,), jnp.int32),
        pltpu.VMEM((NPAD,), jnp.int32),
        pltpu.VMEM((NSUB, SEG), jnp.int32),
        pltpu.VMEM((SEG,), jnp.int32),
        pltpu.VMEM_SHARED((NSUB, NPAD), jnp.int32),
        pltpu.VMEM_SHARED((NSUB, NPAD), jnp.int32),
    ],
    compiler_params=_sc_params,
)
def _sc1(src_hbm, dst_hbm, degp_hbm, c0p_hbm,
         srcb, dstb, degl, c0l, redbuf, redout, degstage, c0stage):
    cid = lax.axis_index("c")
    sid = lax.axis_index("s")
    wid = cid * NSUB + sid

    _zero_ref(degl, NPAD // 16)
    _zero_ref(c0l, NPAD // 16)

    ones = jnp.ones((16,), jnp.int32)

    def chunk(k, _):
        base = wid * ECT1 + k * CH
        pltpu.sync_copy(src_hbm.at[pl.ds(base, CH)], srcb)
        pltpu.sync_copy(dst_hbm.at[pl.ds(base, CH)], dstb)

        def grp(g, _):
            s16 = srcb[pl.ds(g * 16, 16)]
            d16 = dstb[pl.ds(g * 16, 16)]
            plsc.addupdate_scatter(degl, [d16], ones)
            plsc.addupdate_scatter(c0l, [s16], ones, mask=d16 == 0)
            return 0

        lax.fori_loop(0, CH // 16, grp, 0)
        return 0

    lax.fori_loop(0, ECT1 // CH, chunk, 0)

    pltpu.sync_copy(degl, degstage.at[sid])
    pltpu.sync_copy(c0l, c0stage.at[sid])
    plsc.subcore_barrier()

    for stage, out_hbm in ((degstage, degp_hbm), (c0stage, c0p_hbm)):
        for t in range(NSUB):
            pltpu.sync_copy(stage.at[t, pl.ds(sid * SEG, SEG)], redbuf.at[t])

        def red(g, _):
            acc = redbuf[0, pl.ds(g * 16, 16)]
            for t in range(1, NSUB):
                acc = acc + redbuf[t, pl.ds(g * 16, 16)]
            redout[pl.ds(g * 16, 16)] = acc
            return 0

        lax.fori_loop(0, SEG // 16, red, 0)
        pltpu.sync_copy(redout, out_hbm.at[cid, pl.ds(sid * SEG, SEG)])


_BM = 1024


def _tc1_body(state_ref, w_ref, degp_ref, h1s_ref, dis_ref):
    deg = (degp_ref[0, :] + degp_ref[1, :] + 1).astype(jnp.float32)
    dis = lax.rsqrt(deg)
    dis_ref[...] = dis[None, :]
    h = jnp.dot(state_ref[...], w_ref[...], preferred_element_type=jnp.float32)
    h1s_ref[...] = h * dis[:, None]


def _tc1(state_p, w1p, degp):
    return pl.pallas_call(
        _tc1_body,
        grid=(NPAD // _BM,),
        in_specs=[
            pl.BlockSpec((_BM, D), lambda i: (i, 0)),
            pl.BlockSpec((D, GP), lambda i: (0, 0)),
            pl.BlockSpec((NCORE, _BM), lambda i: (0, i)),
        ],
        out_specs=[
            pl.BlockSpec((_BM, GP), lambda i: (i, 0)),
            pl.BlockSpec((1, _BM), lambda i: (0, i)),
        ],
        out_shape=[
            jax.ShapeDtypeStruct((NPAD, GP), jnp.float32),
            jax.ShapeDtypeStruct((1, NPAD), jnp.float32),
        ],
    )(state_p, w1p, degp)


@functools.partial(
    pl.kernel,
    out_type=jax.ShapeDtypeStruct((NCORE, GP), jnp.float32),
    mesh=_mesh,
    scratch_types=[
        pltpu.VMEM((NPAD + 16,), jnp.float32),
        pltpu.VMEM((NPAD,), jnp.int32),
        pltpu.VMEM((SEG,), jnp.int32),
        pltpu.VMEM((SEG,), jnp.int32),
        pltpu.VMEM((MCAP,), jnp.int32),
        pltpu.VMEM((MCAP,), jnp.float32),
        pltpu.VMEM((CH,), jnp.int32),
        pltpu.VMEM((CH,), jnp.int32),
        pltpu.VMEM((ACAP,), jnp.int32),
        pltpu.VMEM((ACAP,), jnp.int32),
        pltpu.VMEM((16, GP), jnp.float32),
        pltpu.VMEM((CROWS, GP), jnp.float32),
        pltpu.VMEM((32,), jnp.int32),
        pltpu.VMEM((32,), jnp.float32),
        pltpu.VMEM((GP,), jnp.float32),
        pltpu.VMEM((GP,), jnp.float32),
        pltpu.VMEM((GP,), jnp.float32),
        pltpu.VMEM((16,), jnp.int32),
        pltpu.VMEM((16,), jnp.int32),
        pltpu.SMEM((8,), jnp.int32),
        pltpu.VMEM_SHARED((NSUB, CROWS, GP), jnp.float32),
        pltpu.VMEM_SHARED((NPAD,), jnp.int32),
        pltpu.VMEM_SHARED((MCAP,), jnp.int32),
        pltpu.VMEM_SHARED((MCAP,), jnp.float32),
        pltpu.VMEM_SHARED((16,), jnp.int32),
        pltpu.VMEM_SHARED((NSUB, GP), jnp.float32),
        pltpu.SemaphoreType.DMA,
    ],
    compiler_params=_sc_params,
)
def _sc2(src_hbm, dst_hbm, c0p_hbm, dis_hbm, h1s_hbm, b1p_hbm, spart_hbm,
         dis_v, remap_v, ca, cb, anodes_v, wc_v, srcb, dstb, asrc, aq,
         rows16, pre1_local, an8, wc8, hrow, slocal, b1p_v, meta_v, qidx,
         cnt_s, prestage, remap_st, an_st, wc_st, meta_st, sstage, gsem):
    cid = lax.axis_index("c")
    sid = lax.axis_index("s")

    pltpu.sync_copy(dis_hbm, dis_v.at[pl.ds(0, NPAD)])
    dis_v[pl.ds(NPAD, 16)] = jnp.zeros((16,), jnp.float32)
    pltpu.sync_copy(b1p_hbm, b1p_v)
    _zero_ref(slocal, GP // 16)
    iota16 = lax.iota(jnp.int32, 16)

    @pl.when(sid == 0)
    def _():
        def init(g, _):
            remap_v[pl.ds(g * 16, 16)] = jnp.full((16,), -1, jnp.int32)
            return 0

        lax.fori_loop(0, NPAD // 16, init, 0)
        dis0 = dis_v[pl.ds(0, 16)][0]
        cnt_s[0] = 0
        cnt_s[1] = 0

        def seg_body(sg, _):
            pltpu.sync_copy(c0p_hbm.at[0, pl.ds(sg * SEG, SEG)], ca)
            pltpu.sync_copy(c0p_hbm.at[1, pl.ds(sg * SEG, SEG)], cb)

            def comp(g, _):
                c = ca[pl.ds(g * 16, 16)] + cb[pl.ds(g * 16, 16)]
                nodeid = iota16 + sg * SEG + g * 16
                act = jnp.logical_or(c > 0, nodeid == 0)
                acti = jnp.where(act, 1, 0)
                ilane = cnt_s[0] + plsc.cumsum(acti) - acti
                sel = jnp.logical_and(act, (ilane & 1) == cid)
                wv = (dis0 * dis_v[pl.ds(sg * SEG + g * 16, 16)]
                      * c.astype(jnp.float32)
                      + jnp.where(nodeid == 0, dis0 * dis0, 0.0))
                cp = cnt_s[1]
                plsc.store_compressed(anodes_v.at[pl.ds(cp, 16)], nodeid,
                                      mask=sel)
                plsc.store_compressed(wc_v.at[pl.ds(cp, 16)], wv, mask=sel)
                plsc.store_scatter(remap_v, [nodeid], ilane, mask=act)
                cnt_s[0] = cnt_s[0] + _popcount(act)
                cnt_s[1] = cp + _popcount(sel)
                return 0

            lax.fori_loop(0, SEG // 16, comp, 0)
            return 0

        lax.fori_loop(0, NPAD // SEG, seg_body, 0)
        meta_v[pl.ds(0, 16)] = jnp.where(iota16 == 0, cnt_s[1], cnt_s[0])
        pltpu.sync_copy(remap_v, remap_st)
        pltpu.sync_copy(anodes_v, an_st)
        pltpu.sync_copy(wc_v, wc_st)
        pltpu.sync_copy(meta_v, meta_st)

    plsc.subcore_barrier()
    pltpu.sync_copy(remap_st, remap_v)
    pltpu.sync_copy(meta_st, meta_v)
    mc = meta_v[pl.ds(0, 16)][0]
    nchunk = (mc + C - 1) // C

    def flush():
        cnt = cnt_s[2]
        asrc[pl.ds(cnt, 16)] = jnp.zeros((16,), jnp.int32)
        aq[pl.ds(cnt, 16)] = jnp.full((16,), C, jnp.int32)
        nb = (cnt + 15) // 16

        def bat(b, _):
            iv = asrc[pl.ds(b * 16, 16)]
            pltpu.async_copy(h1s_hbm.at[iv], rows16, gsem).wait()
            def rowadd(r, _):
                qsc = aq[pl.ds(b * 16 + r, 16)][0]
                for g in range(GP // 16):
                    sl = pl.ds(g * 16, 16)
                    pre1_local[qsc, sl] = pre1_local[qsc, sl] + rows16[r, sl]
                return 0

            lax.fori_loop(0, 16, rowadd, 0)
            return 0

        lax.fori_loop(0, nb, bat, 0)
        cnt_s[2] = 0

    def chunk_body(k, _):
        def zp(r, _):
            for g in range(GP // 16):
                pre1_local[r, pl.ds(g * 16, 16)] = jnp.zeros((16,),
                                                             jnp.float32)
            return 0

        lax.fori_loop(0, CROWS, zp, 0)

        cnt_s[2] = 0

        def echunk(ek, _):
            base = sid * ECT2 + ek * CH
            pltpu.sync_copy(src_hbm.at[pl.ds(base, CH)], srcb)
            pltpu.sync_copy(dst_hbm.at[pl.ds(base, CH)], dstb)

            def grp(g, _):
                s16 = srcb[pl.ds(g * 16, 16)]
                d16 = dstb[pl.ds(g * 16, 16)]
                ri = plsc.load_gather(remap_v, [d16])
                q = (ri >> 1) - k * C
                m = jnp.logical_and(
                    jnp.logical_and(ri >= 0, (ri & 1) == cid),
                    jnp.logical_and(q >= 0, q < C))
                cnt = cnt_s[2]
                plsc.store_compressed(asrc.at[pl.ds(cnt, 16)], s16, mask=m)
                plsc.store_compressed(aq.at[pl.ds(cnt, 16)], q, mask=m)
                cnt_s[2] = cnt + _popcount(m)

                @pl.when(cnt_s[2] >= FLUSH)
                def _():
                    flush()

                return 0

            lax.fori_loop(0, CH // 16, grp, 0)
            return 0

        lax.fori_loop(0, ECT2 // CH, echunk, 0)
        flush()
        pltpu.sync_copy(pre1_local, prestage.at[sid])
        plsc.subcore_barrier()

        rem = (sid % 2) * 4
        abase = pl.multiple_of(k * C + sid * 4 - rem, 8)
        pltpu.sync_copy(an_st.at[pl.ds(abase, 16)], an8.at[pl.ds(0, 16)])
        pltpu.sync_copy(wc_st.at[pl.ds(abase, 16)], wc8.at[pl.ds(0, 16)])

        def fin(t, _):
            q = sid * 4 + t
            p = k * C + q

            @pl.when(jnp.logical_and(q < C, p < mc))
            def _():
                for tt in range(NSUB):
                    pltpu.sync_copy(prestage.at[tt, q], rows16.at[tt])
                j = an8[pl.ds(rem + t, 16)][0]
                wcv = wc8[pl.ds(rem + t, 16)][0]
                dj = dis_v[pl.ds(j, 16)][0]
                pltpu.sync_copy(h1s_hbm.at[j], hrow)
                for g in range(GP // 16):
                    sl = pl.ds(g * 16, 16)
                    acc = rows16[0, sl]
                    for tt in range(1, NSUB):
                        acc = acc + rows16[tt, sl]
                    val = dj * (acc + hrow[sl]) + b1p_v[sl]
                    slocal[sl] = slocal[sl] + jnp.maximum(val, 0.0) * wcv

            return 0

        lax.fori_loop(0, 4, fin, 0)
        plsc.subcore_barrier()
        return 0

    lax.fori_loop(0, nchunk, chunk_body, 0)

    pltpu.sync_copy(slocal, sstage.at[sid])
    plsc.subcore_barrier()

    @pl.when(sid == 0)
    def _():
        pltpu.sync_copy(sstage, rows16)
        for g in range(GP // 16):
            sl = pl.ds(g * 16, 16)
            acc = rows16[0, sl]
            for t in range(1, NSUB):
                acc = acc + rows16[t, sl]
            slocal[sl] = acc
        pltpu.sync_copy(slocal, spart_hbm.at[cid])


def _head_body(sp_ref, w2_ref, b2_ref, f1_ref, fb1_ref, f2_ref, fb2_ref,
               f3_ref, fb3_ref, o_ref):
    s = sp_ref[0, :] + sp_ref[1, :]
    sm = jnp.broadcast_to(s[None, :], (8, GP))
    v = jnp.maximum(jnp.dot(sm, w2_ref[...],
                            preferred_element_type=jnp.float32)
                    + b2_ref[...][None, :], 0.0)
    v = jnp.maximum(jnp.dot(v, f1_ref[...],
                            preferred_element_type=jnp.float32)
                    + fb1_ref[...][None, :], 0.0)
    v = jnp.maximum(jnp.dot(v, f2_ref[...],
                            preferred_element_type=jnp.float32)
                    + fb2_ref[...][None, :], 0.0)
    v = jnp.maximum(jnp.dot(v, f3_ref[...],
                            preferred_element_type=jnp.float32)
                    + fb3_ref[...][None, :], 0.0)
    o_ref[...] = v


def _head(spart, w2p, b2p, f1p, fb1, f2, fb2, f3p, fb3p):
    return pl.pallas_call(
        _head_body,
        out_shape=jax.ShapeDtypeStruct((8, 128), jnp.float32),
    )(spart, w2p, b2p, f1p, fb1, f2, fb2, f3p, fb3p)


def kernel(state, adj, W1, b1, W2, b2, fW1, fb1, fW2, fb2, fW3, fb3):
    src = adj[0]
    dst = adj[1]
    degp, c0p = _sc1(src, dst)

    state_p = jnp.pad(state, ((0, NPAD - N), (0, 0)))
    w1p = jnp.pad(W1, ((0, 0), (0, GP - G)))
    h1s, dis2 = _tc1(state_p, w1p, degp)
    dis = dis2.reshape(NPAD)

    b1p = jnp.pad(b1, (0, GP - G))
    spart = _sc2(src, dst, c0p, dis, h1s, b1p)

    w2p = jnp.pad(W2, ((0, GP - G), (0, GP - G)))
    b2p = jnp.pad(b2, (0, GP - G))
    f1p = jnp.pad(fW1, ((0, GP - G), (0, 0)))
    f3p = jnp.pad(fW3, ((0, 0), (0, 128 - A)))
    fb3p = jnp.pad(fb3, (0, 128 - A))
    o = _head(spart, w2p, b2p, f1p, fb1, fW2, fb2, f3p, fb3p)
    return o[0, :A]


# speedup vs baseline: 79.5752x; 1.0789x over previous
import functools

import jax
import jax.numpy as jnp
from jax import lax
from jax.experimental import pallas as pl
from jax.experimental.pallas import tpu as pltpu, tpu_sc as plsc

N = 10000
E = 320000
D = 128
G = 441
F1 = 256
F2 = 256
A = 10

NPAD = 10240
GP = 512
SEG = NPAD // 16
NSUB = 16
NCORE = 2
ECT1 = E // 32
ECT2 = E // 16
CH = 2000
C = 56
CROWS = 64
MCAP = NPAD // 2 + 64
FLUSH = 512
ACAP = FLUSH + 192
NX = 128

_mesh = plsc.VectorSubcoreMesh(core_axis_name="c", subcore_axis_name="s",
                               num_cores=NCORE, num_subcores=NSUB)
_sc_params = pltpu.CompilerParams(needs_layout_passes=False)

_I16Z = functools.partial(jnp.zeros, (16,))


def _zero_ref(ref, n16):
    def body(g, _):
        ref[pl.ds(g * 16, 16)] = jnp.zeros((16,), ref.dtype)
        return 0
    lax.fori_loop(0, n16, body, 0)


def _popcount(m):
    return plsc.all_reduce_population_count(m)[0]


@functools.partial(
    pl.kernel,
    out_type=(jax.ShapeDtypeStruct((NCORE, NPAD), jnp.int32),
              jax.ShapeDtypeStruct((NCORE, NPAD), jnp.int32)),
    mesh=_mesh,
    scratch_types=[
        pltpu.VMEM((CH,), jnp.int32),
        pltpu.VMEM((CH,), jnp.int32),
        pltpu.VMEM((NPAD,), jnp.int32),
        pltpu.VMEM((NPAD,), jnp.int32),
        pltpu.VMEM((NSUB, SEG), jnp.int32),
        pltpu.VMEM((SEG,), jnp.int32),
        pltpu.VMEM_SHARED((NSUB, NPAD), jnp.int32),
        pltpu.VMEM_SHARED((NSUB, NPAD), jnp.int32),
    ],
    compiler_params=_sc_params,
)
def _sc1(src_hbm, dst_hbm, degp_hbm, c0p_hbm,
         srcb, dstb, degl, c0l, redbuf, redout, degstage, c0stage):
    cid = lax.axis_index("c")
    sid = lax.axis_index("s")
    wid = cid * NSUB + sid

    _zero_ref(degl, NPAD // 16)
    _zero_ref(c0l, NPAD // 16)

    ones = jnp.ones((16,), jnp.int32)

    def chunk(k, _):
        base = wid * ECT1 + k * CH
        pltpu.sync_copy(src_hbm.at[pl.ds(base, CH)], srcb)
        pltpu.sync_copy(dst_hbm.at[pl.ds(base, CH)], dstb)

        def grp(g, _):
            s16 = srcb[pl.ds(g * 16, 16)]
            d16 = dstb[pl.ds(g * 16, 16)]
            plsc.addupdate_scatter(degl, [d16], ones)
            plsc.addupdate_scatter(c0l, [s16], ones, mask=d16 == 0)
            return 0

        lax.fori_loop(0, CH // 16, grp, 0)
        return 0

    lax.fori_loop(0, ECT1 // CH, chunk, 0)

    pltpu.sync_copy(degl, degstage.at[sid])
    pltpu.sync_copy(c0l, c0stage.at[sid])
    plsc.subcore_barrier()

    for stage, out_hbm in ((degstage, degp_hbm), (c0stage, c0p_hbm)):
        for t in range(NSUB):
            pltpu.sync_copy(stage.at[t, pl.ds(sid * SEG, SEG)], redbuf.at[t])

        def red(g, _):
            acc = redbuf[0, pl.ds(g * 16, 16)]
            for t in range(1, NSUB):
                acc = acc + redbuf[t, pl.ds(g * 16, 16)]
            redout[pl.ds(g * 16, 16)] = acc
            return 0

        lax.fori_loop(0, SEG // 16, red, 0)
        pltpu.sync_copy(redout, out_hbm.at[cid, pl.ds(sid * SEG, SEG)])


_BM = 1024


def _tc1_body(state_ref, w_ref, degp_ref, h1s_ref, dis_ref):
    deg = (degp_ref[0, :] + degp_ref[1, :] + 1).astype(jnp.float32)
    dis = lax.rsqrt(deg)
    dis_ref[...] = dis[None, :]
    h = jnp.dot(state_ref[...], w_ref[...], preferred_element_type=jnp.float32)
    h1s_ref[...] = h * dis[:, None]


def _tc1(state_p, w1p, degp):
    return pl.pallas_call(
        _tc1_body,
        grid=(NPAD // _BM,),
        in_specs=[
            pl.BlockSpec((_BM, D), lambda i: (i, 0)),
            pl.BlockSpec((D, GP), lambda i: (0, 0)),
            pl.BlockSpec((NCORE, _BM), lambda i: (0, i)),
        ],
        out_specs=[
            pl.BlockSpec((_BM, GP), lambda i: (i, 0)),
            pl.BlockSpec((1, _BM), lambda i: (0, i)),
        ],
        out_shape=[
            jax.ShapeDtypeStruct((NPAD, GP), jnp.float32),
            jax.ShapeDtypeStruct((1, NPAD), jnp.float32),
        ],
    )(state_p, w1p, degp)


@functools.partial(
    pl.kernel,
    out_type=(jax.ShapeDtypeStruct((NX, GP), jnp.float32),
              jax.ShapeDtypeStruct((NX, 16), jnp.float32),
              jax.ShapeDtypeStruct((NCORE, GP), jnp.float32)),
    mesh=_mesh,
    scratch_types=[
        pltpu.VMEM((NPAD + 16,), jnp.float32),
        pltpu.VMEM((NPAD,), jnp.int32),
        pltpu.VMEM((SEG,), jnp.int32),
        pltpu.VMEM((SEG,), jnp.int32),
        pltpu.VMEM((MCAP,), jnp.int32),
        pltpu.VMEM((MCAP,), jnp.float32),
        pltpu.VMEM((CH,), jnp.int32),
        pltpu.VMEM((CH,), jnp.int32),
        pltpu.VMEM((ACAP,), jnp.int32),
        pltpu.VMEM((16, GP), jnp.float32),
        pltpu.VMEM((CROWS, GP), jnp.float32),
        pltpu.VMEM((32,), jnp.int32),
        pltpu.VMEM((32,), jnp.float32),
        pltpu.VMEM((GP,), jnp.float32),
        pltpu.VMEM((GP,), jnp.float32),
        pltpu.VMEM((16,), jnp.float32),
        pltpu.VMEM((GP,), jnp.float32),
        pltpu.VMEM((GP,), jnp.float32),
        pltpu.VMEM((16,), jnp.int32),
        pltpu.VMEM((16,), jnp.int32),
        pltpu.SMEM((8,), jnp.int32),
        pltpu.VMEM_SHARED((NSUB, CROWS, GP), jnp.float32),
        pltpu.VMEM_SHARED((NPAD,), jnp.int32),
        pltpu.VMEM_SHARED((MCAP,), jnp.int32),
        pltpu.VMEM_SHARED((MCAP,), jnp.float32),
        pltpu.VMEM_SHARED((16,), jnp.int32),
        pltpu.VMEM_SHARED((NSUB, GP), jnp.float32),
        pltpu.SemaphoreType.DMA,
    ],
    compiler_params=_sc_params,
)
def _sc2(src_hbm, dst_hbm, c0p_hbm, dis_hbm, h1s_hbm, b1p_hbm,
         x_hbm, xw_hbm, sovf_hbm,
         dis_v, remap_v, ca, cb, anodes_v, wc_v, srcb, dstb, apack,
         rows16, pre1_local, an8, wc8, hrow, xrow, wbuf, slocal, b1p_v,
         meta_v, qidx, cnt_s, prestage, remap_st, an_st, wc_st, meta_st,
         sstage, gsem):
    cid = lax.axis_index("c")
    sid = lax.axis_index("s")

    pltpu.sync_copy(dis_hbm, dis_v.at[pl.ds(0, NPAD)])
    dis_v[pl.ds(NPAD, 16)] = jnp.zeros((16,), jnp.float32)
    pltpu.sync_copy(b1p_hbm, b1p_v)
    _zero_ref(slocal, GP // 16)
    iota16 = lax.iota(jnp.int32, 16)

    @pl.when(sid == 0)
    def _():
        def init(g, _):
            remap_v[pl.ds(g * 16, 16)] = jnp.full((16,), -1, jnp.int32)
            return 0

        lax.fori_loop(0, NPAD // 16, init, 0)
        dis0 = dis_v[pl.ds(0, 16)][0]

        def seg_body(sg, cnts):
            pltpu.sync_copy(c0p_hbm.at[0, pl.ds(sg * SEG, SEG)], ca)
            pltpu.sync_copy(c0p_hbm.at[1, pl.ds(sg * SEG, SEG)], cb)

            def comp(g, cnts):
                cm, cp = cnts
                c = ca[pl.ds(g * 16, 16)] + cb[pl.ds(g * 16, 16)]
                nodeid = iota16 + sg * SEG + g * 16
                act = jnp.logical_or(c > 0, nodeid == 0)
                acti = jnp.where(act, 1, 0)
                ilane = cm + plsc.cumsum(acti) - acti
                sel = jnp.logical_and(act, (ilane & 1) == cid)
                wv = (dis0 * dis_v[pl.ds(sg * SEG + g * 16, 16)]
                      * c.astype(jnp.float32)
                      + jnp.where(nodeid == 0, dis0 * dis0, 0.0))
                plsc.store_compressed(anodes_v.at[pl.ds(cp, 16)], nodeid,
                                      mask=sel)
                plsc.store_compressed(wc_v.at[pl.ds(cp, 16)], wv, mask=sel)
                plsc.store_scatter(remap_v, [nodeid], ilane >> 1, mask=sel)
                return (cm + _popcount(act), cp + _popcount(sel))

            return lax.fori_loop(0, SEG // 16, comp, cnts)

        cm, cp = lax.fori_loop(0, NPAD // SEG, seg_body, (0, 0))
        meta_v[pl.ds(0, 16)] = jnp.where(iota16 == 0, cp, cm)
        pltpu.sync_copy(remap_v, remap_st)
        pltpu.sync_copy(anodes_v, an_st)
        pltpu.sync_copy(wc_v, wc_st)
        pltpu.sync_copy(meta_v, meta_st)

    _zero_ref(xrow, GP // 16)
    wbuf[...] = jnp.zeros((16,), jnp.float32)
    for t4 in range(4):
        prow0 = sid * 4 + t4
        pltpu.sync_copy(xrow, x_hbm.at[2 * prow0 + cid])
        pltpu.sync_copy(wbuf, xw_hbm.at[2 * prow0 + cid])

    plsc.subcore_barrier()
    pltpu.sync_copy(remap_st, remap_v)
    pltpu.sync_copy(meta_st, meta_v)
    mc = meta_v[pl.ds(0, 16)][0]
    nchunk = (mc + C - 1) // C

    def flush():
        cnt = cnt_s[2]
        apack[pl.ds(cnt, 16)] = jnp.full((16,), C << 14, jnp.int32)
        nb = (cnt + 15) // 16

        def bat(b, _):
            pk = apack[pl.ds(b * 16, 16)]
            iv = pk & 0x3FFF
            pltpu.async_copy(h1s_hbm.at[iv], rows16, gsem).wait()

            def rowadd(r, _):
                qsc = apack[pl.ds(b * 16 + r, 16)][0] >> 14
                for g in range(GP // 16):
                    sl = pl.ds(g * 16, 16)
                    pre1_local[qsc, sl] = pre1_local[qsc, sl] + rows16[r, sl]
                return 0

            lax.fori_loop(0, 16, rowadd, 0)
            return 0

        lax.fori_loop(0, nb, bat, 0)
        cnt_s[2] = 0

    def chunk_body(k, _):
        def zp(r, _):
            for g in range(GP // 16):
                pre1_local[r, pl.ds(g * 16, 16)] = jnp.zeros((16,),
                                                             jnp.float32)
            return 0

        lax.fori_loop(0, CROWS, zp, 0)

        cnt_s[2] = 0

        def echunk(ek, _):
            base = sid * ECT2 + ek * CH
            pltpu.sync_copy(src_hbm.at[pl.ds(base, CH)], srcb)
            pltpu.sync_copy(dst_hbm.at[pl.ds(base, CH)], dstb)

            def sgrp(gg, cnt):
                for u in range(5):
                    g = gg * 5 + u
                    s16 = srcb[pl.ds(g * 16, 16)]
                    d16 = dstb[pl.ds(g * 16, 16)]
                    ri = plsc.load_gather(remap_v, [d16])
                    q = ri - k * C
                    m = jnp.logical_and(q >= 0, q < C)
                    pk = s16 | (q << 14)
                    plsc.store_compressed(apack.at[pl.ds(cnt, 16)], pk,
                                          mask=m)
                    cnt = cnt + _popcount(m)
                cnt_s[2] = cnt

                @pl.when(cnt >= FLUSH)
                def _():
                    flush()

                return cnt_s[2]

            cnt0 = cnt_s[2]
            cnt1 = lax.fori_loop(0, CH // 80, sgrp, cnt0)
            cnt_s[2] = cnt1
            return 0

        lax.fori_loop(0, ECT2 // CH, echunk, 0)
        flush()
        pltpu.sync_copy(pre1_local, prestage.at[sid])
        plsc.subcore_barrier()

        rem = (sid % 2) * 4
        abase = pl.multiple_of(k * C + sid * 4 - rem, 8)
        pltpu.sync_copy(an_st.at[pl.ds(abase, 16)], an8.at[pl.ds(0, 16)])
        pltpu.sync_copy(wc_st.at[pl.ds(abase, 16)], wc8.at[pl.ds(0, 16)])

        def fin(t, _):
            q = sid * 4 + t
            p = k * C + q

            @pl.when(jnp.logical_and(q < C, p < mc))
            def _():
                for tt in range(NSUB):
                    pltpu.sync_copy(prestage.at[tt, q], rows16.at[tt])
                j = an8[pl.ds(rem + t, 16)][0]
                wcv = wc8[pl.ds(rem + t, 16)][0]
                dj = dis_v[pl.ds(j, 16)][0]
                pltpu.sync_copy(h1s_hbm.at[j], hrow)
                for g in range(GP // 16):
                    sl = pl.ds(g * 16, 16)
                    acc = rows16[0, sl]
                    for tt in range(1, NSUB):
                        acc = acc + rows16[tt, sl]
                    val = dj * (acc + hrow[sl]) + b1p_v[sl]
                    xrow[sl] = jnp.maximum(val, 0.0)
                gi = 2 * p + cid

                @pl.when(gi < NX)
                def _():
                    wbuf[...] = jnp.where(iota16 == 0, wcv, 0.0)
                    pltpu.sync_copy(xrow, x_hbm.at[gi])
                    pltpu.sync_copy(wbuf, xw_hbm.at[gi])

                @pl.when(gi >= NX)
                def _():
                    for g in range(GP // 16):
                        sl = pl.ds(g * 16, 16)
                        slocal[sl] = slocal[sl] + xrow[sl] * wcv

            return 0

        lax.fori_loop(0, 4, fin, 0)
        plsc.subcore_barrier()
        return 0

    lax.fori_loop(0, nchunk, chunk_body, 0)

    pltpu.sync_copy(slocal, sstage.at[sid])
    plsc.subcore_barrier()

    @pl.when(sid == 0)
    def _():
        pltpu.sync_copy(sstage, rows16)
        for g in range(GP // 16):
            sl = pl.ds(g * 16, 16)
            acc = rows16[0, sl]
            for t in range(1, NSUB):
                acc = acc + rows16[t, sl]
            slocal[sl] = acc
        pltpu.sync_copy(slocal, sovf_hbm.at[cid])


def _head_body(x_ref, xw_ref, sovf_ref, w2_ref, b2_ref, f1_ref, fb1_ref,
               f2_ref, fb2_ref, f3_ref, fb3_ref, o_ref):
    h2 = jnp.dot(x_ref[...], w2_ref[...], preferred_element_type=jnp.float32)
    wv = xw_ref[...][:, 0]
    m0 = jnp.sum(wv[:, None] * h2, axis=0)
    sv = sovf_ref[0, :] + sovf_ref[1, :]
    svm = jnp.broadcast_to(sv[None, :], (8, GP))
    movf = jnp.dot(svm, w2_ref[...], preferred_element_type=jnp.float32)[0, :]
    s = m0 + movf
    sm = jnp.broadcast_to(s[None, :], (8, GP))
    v = jnp.maximum(sm + b2_ref[...][None, :], 0.0)
    v = jnp.maximum(jnp.dot(v, f1_ref[...],
                            preferred_element_type=jnp.float32)
                    + fb1_ref[...][None, :], 0.0)
    v = jnp.maximum(jnp.dot(v, f2_ref[...],
                            preferred_element_type=jnp.float32)
                    + fb2_ref[...][None, :], 0.0)
    v = jnp.maximum(jnp.dot(v, f3_ref[...],
                            preferred_element_type=jnp.float32)
                    + fb3_ref[...][None, :], 0.0)
    o_ref[...] = v


def _head(x, xw, sovf, w2p, b2p, f1p, fb1, f2, fb2, f3p, fb3p):
    return pl.pallas_call(
        _head_body,
        out_shape=jax.ShapeDtypeStruct((8, 128), jnp.float32),
    )(x, xw, sovf, w2p, b2p, f1p, fb1, f2, fb2, f3p, fb3p)


def kernel(state, adj, W1, b1, W2, b2, fW1, fb1, fW2, fb2, fW3, fb3):
    src = adj[0]
    dst = adj[1]
    degp, c0p = _sc1(src, dst)

    state_p = jnp.pad(state, ((0, NPAD - N), (0, 0)))
    w1p = jnp.pad(W1, ((0, 0), (0, GP - G)))
    h1s, dis2 = _tc1(state_p, w1p, degp)
    dis = dis2.reshape(NPAD)

    b1p = jnp.pad(b1, (0, GP - G))
    x, xw, sovf = _sc2(src, dst, c0p, dis, h1s, b1p)

    w2p = jnp.pad(W2, ((0, GP - G), (0, GP - G)))
    b2p = jnp.pad(b2, (0, GP - G))
    f1p = jnp.pad(fW1, ((0, GP - G), (0, 0)))
    f3p = jnp.pad(fW3, ((0, 0), (0, 128 - A)))
    fb3p = jnp.pad(fb3, (0, 128 - A))
    o = _head(x, xw, sovf, w2p, b2p, f1p, fb1, fW2, fb2, f3p, fb3p)
    return o[0, :A]


# speedup vs baseline: 93.9191x; 1.1803x over previous
import functools

import jax
import jax.numpy as jnp
from jax import lax
from jax.experimental import pallas as pl
from jax.experimental.pallas import tpu as pltpu, tpu_sc as plsc

N = 10000
E = 320000
D = 128
G = 441
F1 = 256
F2 = 256
A = 10

NPAD = 10240
GP = 512
SEG = NPAD // 16
NSUB = 16
NCORE = 2
ECT1 = E // 32
ECT2 = E // 16
CH = 2000
C = 40
CROWS = 48
MCAP = NPAD // 2 + 64
FLUSH = 512
ACAP = FLUSH + 192
NX = 128

_mesh = plsc.VectorSubcoreMesh(core_axis_name="c", subcore_axis_name="s",
                               num_cores=NCORE, num_subcores=NSUB)
_sc_params = pltpu.CompilerParams(needs_layout_passes=False)

_I16Z = functools.partial(jnp.zeros, (16,))


def _zero_ref(ref, n16):
    def body(g, _):
        ref[pl.ds(g * 16, 16)] = jnp.zeros((16,), ref.dtype)
        return 0
    lax.fori_loop(0, n16, body, 0)


def _popcount(m):
    return plsc.all_reduce_population_count(m)[0]


@functools.partial(
    pl.kernel,
    out_type=(jax.ShapeDtypeStruct((NCORE, NPAD), jnp.int32),
              jax.ShapeDtypeStruct((NCORE, NPAD), jnp.int32)),
    mesh=_mesh,
    scratch_types=[
        pltpu.VMEM((CH,), jnp.int32),
        pltpu.VMEM((CH,), jnp.int32),
        pltpu.VMEM((NPAD,), jnp.int32),
        pltpu.VMEM((NPAD,), jnp.int32),
        pltpu.VMEM((NSUB, SEG), jnp.int32),
        pltpu.VMEM((SEG,), jnp.int32),
        pltpu.VMEM_SHARED((NSUB, NPAD), jnp.int32),
        pltpu.VMEM_SHARED((NSUB, NPAD), jnp.int32),
    ],
    compiler_params=_sc_params,
)
def _sc1(src_hbm, dst_hbm, degp_hbm, c0p_hbm,
         srcb, dstb, degl, c0l, redbuf, redout, degstage, c0stage):
    cid = lax.axis_index("c")
    sid = lax.axis_index("s")
    wid = cid * NSUB + sid

    _zero_ref(degl, NPAD // 16)
    _zero_ref(c0l, NPAD // 16)

    ones = jnp.ones((16,), jnp.int32)

    def chunk(k, _):
        base = wid * ECT1 + k * CH
        pltpu.sync_copy(src_hbm.at[pl.ds(base, CH)], srcb)
        pltpu.sync_copy(dst_hbm.at[pl.ds(base, CH)], dstb)

        def grp(g, _):
            s16 = srcb[pl.ds(g * 16, 16)]
            d16 = dstb[pl.ds(g * 16, 16)]
            plsc.addupdate_scatter(degl, [d16], ones)
            plsc.addupdate_scatter(c0l, [s16], ones, mask=d16 == 0)
            return 0

        lax.fori_loop(0, CH // 16, grp, 0)
        return 0

    lax.fori_loop(0, ECT1 // CH, chunk, 0)

    pltpu.sync_copy(degl, degstage.at[sid])
    pltpu.sync_copy(c0l, c0stage.at[sid])
    plsc.subcore_barrier()

    for stage, out_hbm in ((degstage, degp_hbm), (c0stage, c0p_hbm)):
        for t in range(NSUB):
            pltpu.sync_copy(stage.at[t, pl.ds(sid * SEG, SEG)], redbuf.at[t])

        def red(g, _):
            acc = redbuf[0, pl.ds(g * 16, 16)]
            for t in range(1, NSUB):
                acc = acc + redbuf[t, pl.ds(g * 16, 16)]
            redout[pl.ds(g * 16, 16)] = acc
            return 0

        lax.fori_loop(0, SEG // 16, red, 0)
        pltpu.sync_copy(redout, out_hbm.at[cid, pl.ds(sid * SEG, SEG)])


_BM = 1024


def _tc1_body(state_ref, w_ref, degp_ref, h1s_ref, dis_ref):
    deg = (degp_ref[0, :] + degp_ref[1, :] + 1).astype(jnp.float32)
    dis = lax.rsqrt(deg)
    dis_ref[...] = dis[None, :]
    h = jnp.dot(state_ref[...], w_ref[...], preferred_element_type=jnp.float32)
    h1s_ref[...] = h * dis[:, None]


def _tc1(state_p, w1p, degp):
    return pl.pallas_call(
        _tc1_body,
        grid=(NPAD // _BM,),
        in_specs=[
            pl.BlockSpec((_BM, D), lambda i: (i, 0)),
            pl.BlockSpec((D, GP), lambda i: (0, 0)),
            pl.BlockSpec((NCORE, _BM), lambda i: (0, i)),
        ],
        out_specs=[
            pl.BlockSpec((_BM, GP), lambda i: (i, 0)),
            pl.BlockSpec((1, _BM), lambda i: (0, i)),
        ],
        out_shape=[
            jax.ShapeDtypeStruct((NPAD, GP), jnp.float32),
            jax.ShapeDtypeStruct((1, NPAD), jnp.float32),
        ],
    )(state_p, w1p, degp)


@functools.partial(
    pl.kernel,
    out_type=(jax.ShapeDtypeStruct((NX, GP), jnp.float32),
              jax.ShapeDtypeStruct((NX, 16), jnp.float32),
              jax.ShapeDtypeStruct((NCORE, GP), jnp.float32)),
    mesh=_mesh,
    scratch_types=[
        pltpu.VMEM((NPAD + 16,), jnp.float32),
        pltpu.VMEM((NPAD,), jnp.int32),
        pltpu.VMEM((SEG,), jnp.int32),
        pltpu.VMEM((SEG,), jnp.int32),
        pltpu.VMEM((SEG + 16,), jnp.int32),
        pltpu.VMEM((SEG + 16,), jnp.float32),
        pltpu.VMEM((CH,), jnp.int32),
        pltpu.VMEM((CH,), jnp.int32),
        pltpu.VMEM((ACAP,), jnp.int32),
        pltpu.VMEM((16, GP), jnp.float32),
        pltpu.VMEM((CROWS, GP), jnp.float32),
        pltpu.VMEM((32,), jnp.int32),
        pltpu.VMEM((32,), jnp.float32),
        pltpu.VMEM((GP,), jnp.float32),
        pltpu.VMEM((GP,), jnp.float32),
        pltpu.VMEM((16,), jnp.float32),
        pltpu.VMEM((GP,), jnp.float32),
        pltpu.VMEM((GP,), jnp.float32),
        pltpu.VMEM((16,), jnp.int32),
        pltpu.VMEM((16,), jnp.int32),
        pltpu.VMEM((MCAP,), jnp.int32),
        pltpu.VMEM((MCAP,), jnp.float32),
        pltpu.VMEM((16, 16), jnp.int32),
        pltpu.VMEM_SHARED((NSUB, SEG), jnp.int32),
        pltpu.VMEM_SHARED((NSUB, SEG), jnp.float32),
        pltpu.VMEM_SHARED((NSUB, 16), jnp.int32),
        pltpu.SMEM((8,), jnp.int32),
        pltpu.VMEM_SHARED((NSUB, CROWS, GP), jnp.float32),
        pltpu.VMEM_SHARED((MCAP,), jnp.int32),
        pltpu.VMEM_SHARED((MCAP,), jnp.float32),
        pltpu.VMEM_SHARED((16,), jnp.int32),
        pltpu.VMEM_SHARED((NSUB, GP), jnp.float32),
        pltpu.SemaphoreType.DMA,
    ],
    compiler_params=_sc_params,
)
def _sc2(src_hbm, dst_hbm, c0p_hbm, dis_hbm, h1s_hbm, b1p_hbm,
         x_hbm, xw_hbm, sovf_hbm,
         dis_v, remap_v, ca, cb, anodes_v, wc_v, srcb, dstb, apack,
         rows16, pre1_local, an8, wc8, hrow, xrow, wbuf, slocal, b1p_v,
         meta_v, qidx, an2_v, wc2_v, cnt16_v, ansegs_st, wcsegs_st,
         cntstage, cnt_s, prestage, an_st, wc_st, meta_st,
         sstage, gsem):
    cid = lax.axis_index("c")
    sid = lax.axis_index("s")

    pltpu.sync_copy(dis_hbm, dis_v.at[pl.ds(0, NPAD)])
    dis_v[pl.ds(NPAD, 16)] = jnp.zeros((16,), jnp.float32)
    pltpu.sync_copy(b1p_hbm, b1p_v)
    _zero_ref(slocal, GP // 16)
    iota16 = lax.iota(jnp.int32, 16)

    dis0 = dis_v[pl.ds(0, 16)][0]
    pltpu.sync_copy(c0p_hbm.at[0, pl.ds(sid * SEG, SEG)], ca)
    pltpu.sync_copy(c0p_hbm.at[1, pl.ds(sid * SEG, SEG)], cb)

    def compA(g, cp):
        c = ca[pl.ds(g * 16, 16)] + cb[pl.ds(g * 16, 16)]
        nodeid = iota16 + sid * SEG + g * 16
        act = jnp.logical_or(c > 0, nodeid == 0)
        wv = (dis0 * dis_v[pl.ds(sid * SEG + g * 16, 16)]
              * c.astype(jnp.float32)
              + jnp.where(nodeid == 0, dis0 * dis0, 0.0))
        plsc.store_compressed(anodes_v.at[pl.ds(cp, 16)], nodeid, mask=act)
        plsc.store_compressed(wc_v.at[pl.ds(cp, 16)], wv, mask=act)
        return cp + _popcount(act)

    cnt_seg = lax.fori_loop(0, SEG // 16, compA, 0)
    pltpu.sync_copy(anodes_v.at[pl.ds(0, SEG)], ansegs_st.at[sid])
    pltpu.sync_copy(wc_v.at[pl.ds(0, SEG)], wcsegs_st.at[sid])
    meta_v[pl.ds(0, 16)] = jnp.where(iota16 == 0, cnt_seg, 0)
    pltpu.sync_copy(meta_v, cntstage.at[sid])
    plsc.subcore_barrier()

    @pl.when(sid == 0)
    def _():
        pltpu.sync_copy(cntstage, cnt16_v)

        def mergeseg(t, carry):
            ibase, cp = carry
            cnt_t = cnt16_v[t, pl.ds(0, 16)][0]
            pltpu.sync_copy(ansegs_st.at[t], anodes_v.at[pl.ds(0, SEG)])
            pltpu.sync_copy(wcsegs_st.at[t], wc_v.at[pl.ds(0, SEG)])

            def mg(g, cp2):
                nodes = anodes_v[pl.ds(g * 16, 16)]
                wvs = wc_v[pl.ds(g * 16, 16)]
                off = g * 16 + iota16
                ilane = ibase + off
                sel = jnp.logical_and(off < cnt_t, (ilane & 1) == cid)
                plsc.store_compressed(an2_v.at[pl.ds(cp2, 16)], nodes,
                                      mask=sel)
                plsc.store_compressed(wc2_v.at[pl.ds(cp2, 16)], wvs,
                                      mask=sel)
                return cp2 + _popcount(sel)

            cp = lax.fori_loop(0, (cnt_t + 15) // 16, mg, cp)
            return (ibase + cnt_t, cp)

        cm, cp = lax.fori_loop(0, NSUB, mergeseg, (0, 0))
        meta_v[pl.ds(0, 16)] = jnp.where(iota16 == 0, cp, cm)
        pltpu.sync_copy(an2_v, an_st)
        pltpu.sync_copy(wc2_v, wc_st)
        pltpu.sync_copy(meta_v, meta_st)

    _zero_ref(xrow, GP // 16)
    wbuf[...] = jnp.zeros((16,), jnp.float32)
    for t4 in range(4):
        prow0 = sid * 4 + t4
        pltpu.sync_copy(xrow, x_hbm.at[2 * prow0 + cid])
        pltpu.sync_copy(wbuf, xw_hbm.at[2 * prow0 + cid])

    plsc.subcore_barrier()
    pltpu.sync_copy(meta_st, meta_v)
    mc = meta_v[pl.ds(0, 16)][0]
    nchunk = (mc + C - 1) // C

    def rinit(g, _):
        remap_v[pl.ds(g * 16, 16)] = jnp.full((16,), -1, jnp.int32)
        return 0

    lax.fori_loop(0, NPAD // 16, rinit, 0)

    def rbuild(g, _):
        pltpu.sync_copy(an_st.at[pl.ds(g * 16, 16)], qidx)
        nodes = qidx[...]
        p16 = g * 16 + iota16
        plsc.store_scatter(remap_v, [nodes], p16, mask=p16 < mc)
        return 0

    lax.fori_loop(0, (mc + 15) // 16, rbuild, 0)

    def flush():
        cnt = cnt_s[2]
        apack[pl.ds(cnt, 16)] = jnp.full((16,), C << 14, jnp.int32)
        nb = (cnt + 15) // 16

        def bat(b, _):
            pk = apack[pl.ds(b * 16, 16)]
            iv = pk & 0x3FFF
            pltpu.async_copy(h1s_hbm.at[iv], rows16, gsem).wait()

            def rowadd(r, _):
                qsc = apack[pl.ds(b * 16 + r, 16)][0] >> 14
                for g in range(GP // 16):
                    sl = pl.ds(g * 16, 16)
                    pre1_local[qsc, sl] = pre1_local[qsc, sl] + rows16[r, sl]
                return 0

            lax.fori_loop(0, 16, rowadd, 0)
            return 0

        lax.fori_loop(0, nb, bat, 0)
        cnt_s[2] = 0

    def chunk_body(k, _):
        def zp(r, _):
            for g in range(GP // 16):
                pre1_local[r, pl.ds(g * 16, 16)] = jnp.zeros((16,),
                                                             jnp.float32)
            return 0

        lax.fori_loop(0, CROWS, zp, 0)

        cnt_s[2] = 0

        def echunk(ek, _):
            base = sid * ECT2 + ek * CH
            pltpu.sync_copy(src_hbm.at[pl.ds(base, CH)], srcb)
            pltpu.sync_copy(dst_hbm.at[pl.ds(base, CH)], dstb)

            def sgrp(gg, cnt):
                for u in range(5):
                    g = gg * 5 + u
                    s16 = srcb[pl.ds(g * 16, 16)]
                    d16 = dstb[pl.ds(g * 16, 16)]
                    ri = plsc.load_gather(remap_v, [d16])
                    q = ri - k * C
                    m = jnp.logical_and(q >= 0, q < C)
                    pk = s16 | (q << 14)
                    plsc.store_compressed(apack.at[pl.ds(cnt, 16)], pk,
                                          mask=m)
                    cnt = cnt + _popcount(m)
                cnt_s[2] = cnt

                @pl.when(cnt >= FLUSH)
                def _():
                    flush()

                return cnt_s[2]

            cnt0 = cnt_s[2]
            cnt1 = lax.fori_loop(0, CH // 80, sgrp, cnt0)
            cnt_s[2] = cnt1
            return 0

        lax.fori_loop(0, ECT2 // CH, echunk, 0)
        flush()
        pltpu.sync_copy(pre1_local, prestage.at[sid])
        plsc.subcore_barrier()

        rem = (sid * 3) % 8
        abase = pl.multiple_of(k * C + sid * 3 - rem, 8)
        pltpu.sync_copy(an_st.at[pl.ds(abase, 16)], an8.at[pl.ds(0, 16)])
        pltpu.sync_copy(wc_st.at[pl.ds(abase, 16)], wc8.at[pl.ds(0, 16)])

        def fin(t, _):
            q = sid * 3 + t
            p = k * C + q

            @pl.when(jnp.logical_and(q < C, p < mc))
            def _():
                for tt in range(NSUB):
                    pltpu.sync_copy(prestage.at[tt, q], rows16.at[tt])
                j = an8[pl.ds(rem + t, 16)][0]
                wcv = wc8[pl.ds(rem + t, 16)][0]
                dj = dis_v[pl.ds(j, 16)][0]
                pltpu.sync_copy(h1s_hbm.at[j], hrow)
                for g in range(GP // 16):
                    sl = pl.ds(g * 16, 16)
                    acc = rows16[0, sl]
                    for tt in range(1, NSUB):
                        acc = acc + rows16[tt, sl]
                    val = dj * (acc + hrow[sl]) + b1p_v[sl]
                    xrow[sl] = jnp.maximum(val, 0.0)
                gi = 2 * p + cid

                @pl.when(gi < NX)
                def _():
                    wbuf[...] = jnp.where(iota16 == 0, wcv, 0.0)
                    pltpu.sync_copy(xrow, x_hbm.at[gi])
                    pltpu.sync_copy(wbuf, xw_hbm.at[gi])

                @pl.when(gi >= NX)
                def _():
                    for g in range(GP // 16):
                        sl = pl.ds(g * 16, 16)
                        slocal[sl] = slocal[sl] + xrow[sl] * wcv

            return 0

        lax.fori_loop(0, 3, fin, 0)
        plsc.subcore_barrier()
        return 0

    lax.fori_loop(0, nchunk, chunk_body, 0)

    pltpu.sync_copy(slocal, sstage.at[sid])
    plsc.subcore_barrier()

    @pl.when(sid == 0)
    def _():
        pltpu.sync_copy(sstage, rows16)
        for g in range(GP // 16):
            sl = pl.ds(g * 16, 16)
            acc = rows16[0, sl]
            for t in range(1, NSUB):
                acc = acc + rows16[t, sl]
            slocal[sl] = acc
        pltpu.sync_copy(slocal, sovf_hbm.at[cid])


def _head_body(x_ref, xw_ref, sovf_ref, w2_ref, b2_ref, f1_ref, fb1_ref,
               f2_ref, fb2_ref, f3_ref, fb3_ref, o_ref):
    h2 = jnp.dot(x_ref[...], w2_ref[...], preferred_element_type=jnp.float32)
    wv = xw_ref[...][:, 0]
    m0 = jnp.sum(wv[:, None] * h2, axis=0)
    sv = sovf_ref[0, :] + sovf_ref[1, :]
    svm = jnp.broadcast_to(sv[None, :], (8, GP))
    movf = jnp.dot(svm, w2_ref[...], preferred_element_type=jnp.float32)[0, :]
    s = m0 + movf
    sm = jnp.broadcast_to(s[None, :], (8, GP))
    v = jnp.maximum(sm + b2_ref[...][None, :], 0.0)
    v = jnp.maximum(jnp.dot(v, f1_ref[...],
                            preferred_element_type=jnp.float32)
                    + fb1_ref[...][None, :], 0.0)
    v = jnp.maximum(jnp.dot(v, f2_ref[...],
                            preferred_element_type=jnp.float32)
                    + fb2_ref[...][None, :], 0.0)
    v = jnp.maximum(jnp.dot(v, f3_ref[...],
                            preferred_element_type=jnp.float32)
                    + fb3_ref[...][None, :], 0.0)
    o_ref[...] = v


def _head(x, xw, sovf, w2p, b2p, f1p, fb1, f2, fb2, f3p, fb3p):
    return pl.pallas_call(
        _head_body,
        out_shape=jax.ShapeDtypeStruct((8, 128), jnp.float32),
    )(x, xw, sovf, w2p, b2p, f1p, fb1, f2, fb2, f3p, fb3p)


def kernel(state, adj, W1, b1, W2, b2, fW1, fb1, fW2, fb2, fW3, fb3):
    src = adj[0]
    dst = adj[1]
    degp, c0p = _sc1(src, dst)

    state_p = jnp.pad(state, ((0, NPAD - N), (0, 0)))
    w1p = jnp.pad(W1, ((0, 0), (0, GP - G)))
    h1s, dis2 = _tc1(state_p, w1p, degp)
    dis = dis2.reshape(NPAD)

    b1p = jnp.pad(b1, (0, GP - G))
    x, xw, sovf = _sc2(src, dst, c0p, dis, h1s, b1p)

    w2p = jnp.pad(W2, ((0, GP - G), (0, GP - G)))
    b2p = jnp.pad(b2, (0, GP - G))
    f1p = jnp.pad(fW1, ((0, GP - G), (0, 0)))
    f3p = jnp.pad(fW3, ((0, 0), (0, 128 - A)))
    fb3p = jnp.pad(fb3, (0, 128 - A))
    o = _head(x, xw, sovf, w2p, b2p, f1p, fb1, fW2, fb2, f3p, fb3p)
    return o[0, :A]


# speedup vs baseline: 97.2798x; 1.0358x over previous
import functools

import jax
import jax.numpy as jnp
from jax import lax
from jax.experimental import pallas as pl
from jax.experimental.pallas import tpu as pltpu, tpu_sc as plsc

N = 10000
E = 320000
D = 128
G = 441
F1 = 256
F2 = 256
A = 10

NPAD = 10240
GP = 512
SEG = NPAD // 16
NSUB = 16
NCORE = 2
ECT1 = E // 32
ECT2 = E // 16
CH = 2000
C = 40
CROWS = 48
MCAP = NPAD // 2 + 64
FLUSH = 512
ACAP = FLUSH + 192
NX = 128

_mesh = plsc.VectorSubcoreMesh(core_axis_name="c", subcore_axis_name="s",
                               num_cores=NCORE, num_subcores=NSUB)
_sc_params = pltpu.CompilerParams(needs_layout_passes=False)

_I16Z = functools.partial(jnp.zeros, (16,))


def _zero_ref(ref, n16):
    def body(g, _):
        ref[pl.ds(g * 16, 16)] = jnp.zeros((16,), ref.dtype)
        return 0
    lax.fori_loop(0, n16, body, 0)


def _popcount(m):
    return plsc.all_reduce_population_count(m)[0]


@functools.partial(
    pl.kernel,
    out_type=(jax.ShapeDtypeStruct((NCORE, NPAD), jnp.int32),
              jax.ShapeDtypeStruct((NCORE, NPAD), jnp.int32)),
    mesh=_mesh,
    scratch_types=[
        pltpu.VMEM((CH,), jnp.int32),
        pltpu.VMEM((CH,), jnp.int32),
        pltpu.VMEM((NPAD,), jnp.int32),
        pltpu.VMEM((NPAD,), jnp.int32),
        pltpu.VMEM((NSUB, SEG), jnp.int32),
        pltpu.VMEM((SEG,), jnp.int32),
        pltpu.VMEM_SHARED((NSUB, NPAD), jnp.int32),
        pltpu.VMEM_SHARED((NSUB, NPAD), jnp.int32),
        pltpu.SemaphoreType.DMA,
    ],
    compiler_params=_sc_params,
)
def _sc1(src_hbm, dst_hbm, degp_hbm, c0p_hbm,
         srcb, dstb, degl, c0l, redbuf, redout, degstage, c0stage, rsem):
    cid = lax.axis_index("c")
    sid = lax.axis_index("s")
    wid = cid * NSUB + sid

    _zero_ref(degl, NPAD // 16)
    _zero_ref(c0l, NPAD // 16)

    ones = jnp.ones((16,), jnp.int32)

    def chunk(k, _):
        base = wid * ECT1 + k * CH
        pltpu.sync_copy(src_hbm.at[pl.ds(base, CH)], srcb)
        pltpu.sync_copy(dst_hbm.at[pl.ds(base, CH)], dstb)

        def grp(g, _):
            s16 = srcb[pl.ds(g * 16, 16)]
            d16 = dstb[pl.ds(g * 16, 16)]
            plsc.addupdate_scatter(degl, [d16], ones)
            plsc.addupdate_scatter(c0l, [s16], ones, mask=d16 == 0)
            return 0

        lax.fori_loop(0, CH // 16, grp, 0)
        return 0

    lax.fori_loop(0, ECT1 // CH, chunk, 0)

    pltpu.sync_copy(degl, degstage.at[sid])
    pltpu.sync_copy(c0l, c0stage.at[sid])
    plsc.subcore_barrier()

    for stage, out_hbm in ((degstage, degp_hbm), (c0stage, c0p_hbm)):
        descs = [pltpu.async_copy(stage.at[t, pl.ds(sid * SEG, SEG)],
                                  redbuf.at[t], rsem) for t in range(NSUB)]
        for d in descs:
            d.wait()

        def red(g, _):
            acc = redbuf[0, pl.ds(g * 16, 16)]
            for t in range(1, NSUB):
                acc = acc + redbuf[t, pl.ds(g * 16, 16)]
            redout[pl.ds(g * 16, 16)] = acc
            return 0

        lax.fori_loop(0, SEG // 16, red, 0)
        pltpu.sync_copy(redout, out_hbm.at[cid, pl.ds(sid * SEG, SEG)])


_BM = 1024


def _tc1_body(state_ref, w_ref, degp_ref, h1s_ref, dis_ref):
    deg = (degp_ref[0, :] + degp_ref[1, :] + 1).astype(jnp.float32)
    dis = lax.rsqrt(deg)
    dis_ref[...] = dis[None, :]
    h = jnp.dot(state_ref[...], w_ref[...], preferred_element_type=jnp.float32)
    h1s_ref[...] = h * dis[:, None]


def _tc1(state_p, w1p, degp):
    return pl.pallas_call(
        _tc1_body,
        grid=(NPAD // _BM,),
        in_specs=[
            pl.BlockSpec((_BM, D), lambda i: (i, 0)),
            pl.BlockSpec((D, GP), lambda i: (0, 0)),
            pl.BlockSpec((NCORE, _BM), lambda i: (0, i)),
        ],
        out_specs=[
            pl.BlockSpec((_BM, GP), lambda i: (i, 0)),
            pl.BlockSpec((1, _BM), lambda i: (0, i)),
        ],
        out_shape=[
            jax.ShapeDtypeStruct((NPAD, GP), jnp.float32),
            jax.ShapeDtypeStruct((1, NPAD), jnp.float32),
        ],
    )(state_p, w1p, degp)


@functools.partial(
    pl.kernel,
    out_type=(jax.ShapeDtypeStruct((NX, GP), jnp.float32),
              jax.ShapeDtypeStruct((NX, 16), jnp.float32),
              jax.ShapeDtypeStruct((NCORE, GP), jnp.float32)),
    mesh=_mesh,
    scratch_types=[
        pltpu.VMEM((NPAD + 16,), jnp.float32),
        pltpu.VMEM((NPAD,), jnp.int32),
        pltpu.VMEM((SEG,), jnp.int32),
        pltpu.VMEM((SEG,), jnp.int32),
        pltpu.VMEM((SEG + 16,), jnp.int32),
        pltpu.VMEM((SEG + 16,), jnp.float32),
        pltpu.VMEM((CH,), jnp.int32),
        pltpu.VMEM((CH,), jnp.int32),
        pltpu.VMEM((ACAP,), jnp.int32),
        pltpu.VMEM((16, GP), jnp.float32),
        pltpu.VMEM((CROWS, GP), jnp.float32),
        pltpu.VMEM((32,), jnp.int32),
        pltpu.VMEM((32,), jnp.float32),
        pltpu.VMEM((GP,), jnp.float32),
        pltpu.VMEM((GP,), jnp.float32),
        pltpu.VMEM((16,), jnp.float32),
        pltpu.VMEM((GP,), jnp.float32),
        pltpu.VMEM((GP,), jnp.float32),
        pltpu.VMEM((16,), jnp.int32),
        pltpu.VMEM((16,), jnp.int32),
        pltpu.VMEM((MCAP,), jnp.int32),
        pltpu.VMEM((MCAP,), jnp.float32),
        pltpu.VMEM((16, 16), jnp.int32),
        pltpu.VMEM_SHARED((NSUB, SEG), jnp.int32),
        pltpu.VMEM_SHARED((NSUB, SEG), jnp.float32),
        pltpu.VMEM_SHARED((NSUB, 16), jnp.int32),
        pltpu.SMEM((8,), jnp.int32),
        pltpu.VMEM_SHARED((NSUB, CROWS, GP), jnp.float32),
        pltpu.VMEM_SHARED((MCAP,), jnp.int32),
        pltpu.VMEM_SHARED((MCAP,), jnp.float32),
        pltpu.VMEM_SHARED((16,), jnp.int32),
        pltpu.VMEM_SHARED((NSUB, GP), jnp.float32),
        pltpu.SemaphoreType.DMA,
    ],
    compiler_params=_sc_params,
)
def _sc2(src_hbm, dst_hbm, c0p_hbm, dis_hbm, h1s_hbm, b1p_hbm,
         x_hbm, xw_hbm, sovf_hbm,
         dis_v, remap_v, ca, cb, anodes_v, wc_v, srcb, dstb, apack,
         rows16, pre1_local, an8, wc8, hrow, xrow, wbuf, slocal, b1p_v,
         meta_v, qidx, an2_v, wc2_v, cnt16_v, ansegs_st, wcsegs_st,
         cntstage, cnt_s, prestage, an_st, wc_st, meta_st,
         sstage, gsem):
    cid = lax.axis_index("c")
    sid = lax.axis_index("s")

    pltpu.sync_copy(dis_hbm, dis_v.at[pl.ds(0, NPAD)])
    dis_v[pl.ds(NPAD, 16)] = jnp.zeros((16,), jnp.float32)
    pltpu.sync_copy(b1p_hbm, b1p_v)
    _zero_ref(slocal, GP // 16)
    iota16 = lax.iota(jnp.int32, 16)

    dis0 = dis_v[pl.ds(0, 16)][0]
    pltpu.sync_copy(c0p_hbm.at[0, pl.ds(sid * SEG, SEG)], ca)
    pltpu.sync_copy(c0p_hbm.at[1, pl.ds(sid * SEG, SEG)], cb)

    def compA(g, cp):
        c = ca[pl.ds(g * 16, 16)] + cb[pl.ds(g * 16, 16)]
        nodeid = iota16 + sid * SEG + g * 16
        act = jnp.logical_or(c > 0, nodeid == 0)
        wv = (dis0 * dis_v[pl.ds(sid * SEG + g * 16, 16)]
              * c.astype(jnp.float32)
              + jnp.where(nodeid == 0, dis0 * dis0, 0.0))
        plsc.store_compressed(anodes_v.at[pl.ds(cp, 16)], nodeid, mask=act)
        plsc.store_compressed(wc_v.at[pl.ds(cp, 16)], wv, mask=act)
        return cp + _popcount(act)

    cnt_seg = lax.fori_loop(0, SEG // 16, compA, 0)
    pltpu.sync_copy(anodes_v.at[pl.ds(0, SEG)], ansegs_st.at[sid])
    pltpu.sync_copy(wc_v.at[pl.ds(0, SEG)], wcsegs_st.at[sid])
    meta_v[pl.ds(0, 16)] = jnp.where(iota16 == 0, cnt_seg, 0)
    pltpu.sync_copy(meta_v, cntstage.at[sid])
    plsc.subcore_barrier()

    @pl.when(sid == 0)
    def _():
        pltpu.sync_copy(cntstage, cnt16_v)

        def mergeseg(t, carry):
            ibase, cp = carry
            cnt_t = cnt16_v[t, pl.ds(0, 16)][0]
            pltpu.sync_copy(ansegs_st.at[t], anodes_v.at[pl.ds(0, SEG)])
            pltpu.sync_copy(wcsegs_st.at[t], wc_v.at[pl.ds(0, SEG)])

            def mg(g, cp2):
                nodes = anodes_v[pl.ds(g * 16, 16)]
                wvs = wc_v[pl.ds(g * 16, 16)]
                off = g * 16 + iota16
                ilane = ibase + off
                sel = jnp.logical_and(off < cnt_t, (ilane & 1) == cid)
                plsc.store_compressed(an2_v.at[pl.ds(cp2, 16)], nodes,
                                      mask=sel)
                plsc.store_compressed(wc2_v.at[pl.ds(cp2, 16)], wvs,
                                      mask=sel)
                return cp2 + _popcount(sel)

            cp = lax.fori_loop(0, (cnt_t + 15) // 16, mg, cp)
            return (ibase + cnt_t, cp)

        cm, cp = lax.fori_loop(0, NSUB, mergeseg, (0, 0))
        meta_v[pl.ds(0, 16)] = jnp.where(iota16 == 0, cp, cm)
        pltpu.sync_copy(an2_v, an_st)
        pltpu.sync_copy(wc2_v, wc_st)
        pltpu.sync_copy(meta_v, meta_st)

    _zero_ref(xrow, GP // 16)
    wbuf[...] = jnp.zeros((16,), jnp.float32)
    for t4 in range(4):
        prow0 = sid * 4 + t4
        pltpu.sync_copy(xrow, x_hbm.at[2 * prow0 + cid])
        pltpu.sync_copy(wbuf, xw_hbm.at[2 * prow0 + cid])

    plsc.subcore_barrier()
    pltpu.sync_copy(meta_st, meta_v)
    mc = meta_v[pl.ds(0, 16)][0]
    nchunk = (mc + C - 1) // C

    def rinit(g, _):
        remap_v[pl.ds(g * 16, 16)] = jnp.full((16,), -1, jnp.int32)
        return 0

    lax.fori_loop(0, NPAD // 16, rinit, 0)

    def rbuild(g, _):
        pltpu.sync_copy(an_st.at[pl.ds(g * 16, 16)], qidx)
        nodes = qidx[...]
        p16 = g * 16 + iota16
        plsc.store_scatter(remap_v, [nodes], p16, mask=p16 < mc)
        return 0

    lax.fori_loop(0, (mc + 15) // 16, rbuild, 0)

    def flush():
        cnt = cnt_s[2]
        apack[pl.ds(cnt, 16)] = jnp.full((16,), C << 14, jnp.int32)
        nb = (cnt + 15) // 16

        def bat(b, _):
            pk = apack[pl.ds(b * 16, 16)]
            iv = pk & 0x3FFF
            pltpu.async_copy(h1s_hbm.at[iv], rows16, gsem).wait()

            def rowadd(r, _):
                qsc = apack[pl.ds(b * 16 + r, 16)][0] >> 14
                for g in range(GP // 16):
                    sl = pl.ds(g * 16, 16)
                    pre1_local[qsc, sl] = pre1_local[qsc, sl] + rows16[r, sl]
                return 0

            lax.fori_loop(0, 16, rowadd, 0)
            return 0

        lax.fori_loop(0, nb, bat, 0)
        cnt_s[2] = 0

    def chunk_body(k, _):
        def zp(r, _):
            for g in range(GP // 16):
                pre1_local[r, pl.ds(g * 16, 16)] = jnp.zeros((16,),
                                                             jnp.float32)
            return 0

        lax.fori_loop(0, CROWS, zp, 0)

        cnt_s[2] = 0

        def echunk(ek, _):
            base = sid * ECT2 + ek * CH
            pltpu.sync_copy(src_hbm.at[pl.ds(base, CH)], srcb)
            pltpu.sync_copy(dst_hbm.at[pl.ds(base, CH)], dstb)

            def sgrp(gg, cnt):
                for u in range(5):
                    g = gg * 5 + u
                    s16 = srcb[pl.ds(g * 16, 16)]
                    d16 = dstb[pl.ds(g * 16, 16)]
                    ri = plsc.load_gather(remap_v, [d16])
                    q = ri - k * C
                    m = jnp.logical_and(q >= 0, q < C)
                    pk = s16 | (q << 14)
                    plsc.store_compressed(apack.at[pl.ds(cnt, 16)], pk,
                                          mask=m)
                    cnt = cnt + _popcount(m)
                cnt_s[2] = cnt

                @pl.when(cnt >= FLUSH)
                def _():
                    flush()

                return cnt_s[2]

            cnt0 = cnt_s[2]
            cnt1 = lax.fori_loop(0, CH // 80, sgrp, cnt0)
            cnt_s[2] = cnt1
            return 0

        lax.fori_loop(0, ECT2 // CH, echunk, 0)
        flush()
        pltpu.sync_copy(pre1_local, prestage.at[sid])
        plsc.subcore_barrier()

        rem = (sid * 3) % 8
        abase = pl.multiple_of(k * C + sid * 3 - rem, 8)
        pltpu.sync_copy(an_st.at[pl.ds(abase, 16)], an8.at[pl.ds(0, 16)])
        pltpu.sync_copy(wc_st.at[pl.ds(abase, 16)], wc8.at[pl.ds(0, 16)])

        def fin(t, _):
            q = sid * 3 + t
            p = k * C + q

            @pl.when(jnp.logical_and(q < C, p < mc))
            def _():
                descs = [pltpu.async_copy(prestage.at[tt, q],
                                          rows16.at[tt], gsem)
                         for tt in range(NSUB)]
                for d in descs:
                    d.wait()
                j = an8[pl.ds(rem + t, 16)][0]
                wcv = wc8[pl.ds(rem + t, 16)][0]
                dj = dis_v[pl.ds(j, 16)][0]
                pltpu.sync_copy(h1s_hbm.at[j], hrow)
                for g in range(GP // 16):
                    sl = pl.ds(g * 16, 16)
                    acc = rows16[0, sl]
                    for tt in range(1, NSUB):
                        acc = acc + rows16[tt, sl]
                    val = dj * (acc + hrow[sl]) + b1p_v[sl]
                    xrow[sl] = jnp.maximum(val, 0.0)
                gi = 2 * p + cid

                @pl.when(gi < NX)
                def _():
                    wbuf[...] = jnp.where(iota16 == 0, wcv, 0.0)
                    pltpu.sync_copy(xrow, x_hbm.at[gi])
                    pltpu.sync_copy(wbuf, xw_hbm.at[gi])

                @pl.when(gi >= NX)
                def _():
                    for g in range(GP // 16):
                        sl = pl.ds(g * 16, 16)
                        slocal[sl] = slocal[sl] + xrow[sl] * wcv

            return 0

        lax.fori_loop(0, 3, fin, 0)
        plsc.subcore_barrier()
        return 0

    lax.fori_loop(0, nchunk, chunk_body, 0)

    pltpu.sync_copy(slocal, sstage.at[sid])
    plsc.subcore_barrier()

    @pl.when(sid == 0)
    def _():
        pltpu.sync_copy(sstage, rows16)
        for g in range(GP // 16):
            sl = pl.ds(g * 16, 16)
            acc = rows16[0, sl]
            for t in range(1, NSUB):
                acc = acc + rows16[t, sl]
            slocal[sl] = acc
        pltpu.sync_copy(slocal, sovf_hbm.at[cid])


def _head_body(x_ref, xw_ref, sovf_ref, w2_ref, b2_ref, f1_ref, fb1_ref,
               f2_ref, fb2_ref, f3_ref, fb3_ref, o_ref):
    h2 = jnp.dot(x_ref[...], w2_ref[...], preferred_element_type=jnp.float32)
    wv = xw_ref[...][:, 0]
    m0 = jnp.sum(wv[:, None] * h2, axis=0)
    sv = sovf_ref[0, :] + sovf_ref[1, :]
    svm = jnp.broadcast_to(sv[None, :], (8, GP))
    movf = jnp.dot(svm, w2_ref[...], preferred_element_type=jnp.float32)[0, :]
    s = m0 + movf
    sm = jnp.broadcast_to(s[None, :], (8, GP))
    v = jnp.maximum(sm + b2_ref[...][None, :], 0.0)
    v = jnp.maximum(jnp.dot(v, f1_ref[...],
                            preferred_element_type=jnp.float32)
                    + fb1_ref[...][None, :], 0.0)
    v = jnp.maximum(jnp.dot(v, f2_ref[...],
                            preferred_element_type=jnp.float32)
                    + fb2_ref[...][None, :], 0.0)
    v = jnp.maximum(jnp.dot(v, f3_ref[...],
                            preferred_element_type=jnp.float32)
                    + fb3_ref[...][None, :], 0.0)
    o_ref[...] = v


def _head(x, xw, sovf, w2p, b2p, f1p, fb1, f2, fb2, f3p, fb3p):
    return pl.pallas_call(
        _head_body,
        out_shape=jax.ShapeDtypeStruct((8, 128), jnp.float32),
    )(x, xw, sovf, w2p, b2p, f1p, fb1, f2, fb2, f3p, fb3p)


def kernel(state, adj, W1, b1, W2, b2, fW1, fb1, fW2, fb2, fW3, fb3):
    src = adj[0]
    dst = adj[1]
    degp, c0p = _sc1(src, dst)

    state_p = jnp.pad(state, ((0, NPAD - N), (0, 0)))
    w1p = jnp.pad(W1, ((0, 0), (0, GP - G)))
    h1s, dis2 = _tc1(state_p, w1p, degp)
    dis = dis2.reshape(NPAD)

    b1p = jnp.pad(b1, (0, GP - G))
    x, xw, sovf = _sc2(src, dst, c0p, dis, h1s, b1p)

    w2p = jnp.pad(W2, ((0, GP - G), (0, GP - G)))
    b2p = jnp.pad(b2, (0, GP - G))
    f1p = jnp.pad(fW1, ((0, GP - G), (0, 0)))
    f3p = jnp.pad(fW3, ((0, 0), (0, 128 - A)))
    fb3p = jnp.pad(fb3, (0, 128 - A))
    o = _head(x, xw, sovf, w2p, b2p, f1p, fb1, fW2, fb2, f3p, fb3p)
    return o[0, :A]
